# Initial kernel scaffold; baseline (speedup 1.0000x reference)
import jax
import jax.numpy as jnp
from jax.experimental import pallas as pl


def kernel(x, edge_index, edge_type, edge_attr, W1, W2):
    raise NotImplementedError("write your pallas kernel here")


# speedup vs baseline: 23.2654x; 23.2654x over previous
import functools

import jax
import jax.numpy as jnp
from jax import lax
from jax.experimental import pallas as pl
from jax.experimental.pallas import tpu as pltpu
from jax.experimental.pallas import tpu_sc as plsc

N = 10000
E = 320000
D = 128
H = 32
C = 16
R = 8

NC, NS, L = 2, 16, 16
CH = 128
E1 = ((E + NC * NS * CH - 1) // (NC * NS * CH)) * (NC * NS * CH)
CH_P2 = E1 // (NC * NS * CH)
CH_P1 = E1 // (NS * CH)
CNTP = 96000
ZT = 10
ZR = N // ZT
ZC = 40
CB = 200
YROWS = R * N

_mesh = plsc.VectorSubcoreMesh(
    core_axis_name="c", subcore_axis_name="s", num_cores=NC, num_subcores=NS)
_sc_params = pltpu.CompilerParams(use_tc_tiling_on_sc=False)


def _sc_layer1(ep, yt):
    @functools.partial(
        pl.kernel,
        out_type=(jax.ShapeDtypeStruct((NC, N, H), jnp.float32),
                  jax.ShapeDtypeStruct((E1,), jnp.float32)),
        mesh=_mesh,
        compiler_params=_sc_params,
        scratch_types=[
            pltpu.VMEM_SHARED((CNTP,), jnp.float32),
            pltpu.VMEM_SHARED((N, H), jnp.float32),
            pltpu.VMEM((4, CH), jnp.int32),
            pltpu.VMEM((CH,), jnp.int32),
            pltpu.VMEM((CH,), jnp.int32),
            pltpu.VMEM((CH,), jnp.float32),
            pltpu.VMEM((CH,), jnp.float32),
            pltpu.VMEM((CH,), jnp.float32),
            pltpu.VMEM((CH, H), jnp.float32),
            pltpu.VMEM((1200,), jnp.float32),
            pltpu.VMEM((ZC, H), jnp.float32),
            pltpu.VMEM((CB, H), jnp.float32),
        ],
    )
    def k(ep_h, y_h, out1p_h, a_h,
          cnt_sp, acc_sp, e4_v, gidx_v, cidx_v, cval_v, aval_v,
          ones_v, rows_v, zc_v, zr_v, cb_v):
        c = lax.axis_index("c")
        s = lax.axis_index("s")

        for j in range(CH // L):
            ones_v[pl.ds(j * L, L)] = jnp.ones((L,), jnp.float32)
        for j in range(1200 // L):
            zc_v[pl.ds(j * L, L)] = jnp.zeros((L,), jnp.float32)
        for r in range(ZC):
            zr_v[r, pl.ds(0, L)] = jnp.zeros((L,), jnp.float32)
            zr_v[r, pl.ds(L, L)] = jnp.zeros((L,), jnp.float32)

        for i in range(CNTP // NS // 1200):
            pltpu.sync_copy(zc_v, cnt_sp.at[pl.ds(s * (CNTP // NS) + i * 1200,
                                                  1200)])

        @pl.when(s < ZT)
        def _():
            def zb(i, carry):
                pltpu.sync_copy(zr_v, acc_sp.at[pl.ds(s * ZR + i * ZC, ZC)])
                return carry
            lax.fori_loop(0, ZR // ZC, zb, 0)

        plsc.subcore_barrier()

        def ph1(i, carry):
            base = (s * CH_P1 + i) * CH
            pltpu.sync_copy(ep_h.at[pl.ds(0, 4), pl.ds(base, CH)], e4_v)
            for j in range(CH // L):
                t = e4_v[1, pl.ds(j * L, L)]
                d = e4_v[2, pl.ds(j * L, L)]
                cidx_v[pl.ds(j * L, L)] = t * N + d
            pltpu.sync_copy(ones_v, cnt_sp.at[cidx_v], add=True)
            return carry

        lax.fori_loop(0, CH_P1, ph1, 0)
        plsc.subcore_barrier()

        def ph2(i, carry):
            base = ((c * NS + s) * CH_P2 + i) * CH
            pltpu.sync_copy(ep_h.at[pl.ds(0, 4), pl.ds(base, CH)], e4_v)
            for j in range(CH // L):
                sl = pl.ds(j * L, L)
                t = e4_v[1, sl]
                tn = t * N
                gidx_v[sl] = jnp.minimum(tn + e4_v[0, sl], YROWS - 1)
                cidx_v[sl] = tn + e4_v[2, sl]
            pltpu.sync_copy(cnt_sp.at[cidx_v], cval_v)
            for j in range(CH // L):
                sl = pl.ds(j * L, L)
                w = lax.bitcast_convert_type(e4_v[3, sl], jnp.float32)
                aval_v[sl] = w / jnp.maximum(cval_v[sl], 1.0)
            pltpu.sync_copy(aval_v, a_h.at[pl.ds(base, CH)])
            pltpu.sync_copy(y_h.at[gidx_v], rows_v)
            for j in range(CH // L):
                av = aval_v[pl.ds(j * L, L)]
                for kk in range(L):
                    e = j * L + kk
                    b = av[kk]
                    rows_v[e, pl.ds(0, L)] = rows_v[e, pl.ds(0, L)] * b
                    rows_v[e, pl.ds(L, L)] = rows_v[e, pl.ds(L, L)] * b
            didx = e4_v.at[2]
            pltpu.sync_copy(rows_v, acc_sp.at[didx], add=True)
            return carry

        lax.fori_loop(0, CH_P2, ph2, 0)
        plsc.subcore_barrier()

        @pl.when(s < ZT)
        def _():
            def drain(i, carry):
                off = s * ZR + i * CB
                pltpu.sync_copy(acc_sp.at[pl.ds(off, CB)], cb_v)
                pltpu.sync_copy(cb_v, out1p_h.at[c, pl.ds(off, CB)])
                return carry
            lax.fori_loop(0, ZR // CB, drain, 0)

    return k(ep, yt)


def _sc_layer2(ep, a, zt):
    @functools.partial(
        pl.kernel,
        out_type=jax.ShapeDtypeStruct((NC, N, C), jnp.float32),
        mesh=_mesh,
        compiler_params=_sc_params,
        scratch_types=[
            pltpu.VMEM_SHARED((N, C), jnp.float32),
            pltpu.VMEM((4, CH), jnp.int32),
            pltpu.VMEM((CH,), jnp.int32),
            pltpu.VMEM((CH,), jnp.float32),
            pltpu.VMEM((CH, C), jnp.float32),
            pltpu.VMEM((ZC, C), jnp.float32),
            pltpu.VMEM((CB, C), jnp.float32),
        ],
    )
    def k(ep_h, a_h, z_h, out2p_h, acc_sp, e4_v, gidx_v, aval_v, rows_v,
          zr_v, cb_v):
        c = lax.axis_index("c")
        s = lax.axis_index("s")

        for r in range(ZC):
            zr_v[r, pl.ds(0, L)] = jnp.zeros((L,), jnp.float32)

        @pl.when(s < ZT)
        def _():
            def zb(i, carry):
                pltpu.sync_copy(zr_v, acc_sp.at[pl.ds(s * ZR + i * ZC, ZC)])
                return carry
            lax.fori_loop(0, ZR // ZC, zb, 0)

        plsc.subcore_barrier()

        def ph(i, carry):
            base = ((c * NS + s) * CH_P2 + i) * CH
            pltpu.sync_copy(ep_h.at[pl.ds(0, 4), pl.ds(base, CH)], e4_v)
            pltpu.sync_copy(a_h.at[pl.ds(base, CH)], aval_v)
            for j in range(CH // L):
                sl = pl.ds(j * L, L)
                gidx_v[sl] = jnp.minimum(
                    e4_v[1, sl] * N + e4_v[0, sl], YROWS - 1)
            pltpu.sync_copy(z_h.at[gidx_v], rows_v)
            for j in range(CH // L):
                av = aval_v[pl.ds(j * L, L)]
                for kk in range(L):
                    e = j * L + kk
                    rows_v[e, pl.ds(0, L)] = rows_v[e, pl.ds(0, L)] * av[kk]
            didx = e4_v.at[2]
            pltpu.sync_copy(rows_v, acc_sp.at[didx], add=True)
            return carry

        lax.fori_loop(0, CH_P2, ph, 0)
        plsc.subcore_barrier()

        @pl.when(s < ZT)
        def _():
            def drain(i, carry):
                off = s * ZR + i * CB
                pltpu.sync_copy(acc_sp.at[pl.ds(off, CB)], cb_v)
                pltpu.sync_copy(cb_v, out2p_h.at[c, pl.ds(off, CB)])
                return carry
            lax.fori_loop(0, ZR // CB, drain, 0)

    return k(ep, a, zt)


def _tc_transform1(x, W1):
    def body(x_ref, w_ref, y_ref):
        xv = x_ref[...]
        for r in range(R):
            y_ref[r * N:(r + 1) * N, :] = jnp.dot(
                xv, w_ref[r], preferred_element_type=jnp.float32)

    return pl.pallas_call(
        body,
        out_shape=jax.ShapeDtypeStruct((YROWS, H), jnp.float32),
    )(x, W1)


def _tc_middle(out1p, W2):
    def body(p_ref, w_ref, z_ref):
        h = jnp.maximum(p_ref[0] + p_ref[1], 0.0)
        for r in range(R):
            z_ref[r * N:(r + 1) * N, :] = jnp.dot(
                h, w_ref[r], preferred_element_type=jnp.float32)

    return pl.pallas_call(
        body,
        out_shape=jax.ShapeDtypeStruct((YROWS, C), jnp.float32),
    )(out1p, W2)


def _tc_final(out2p):
    def body(p_ref, o_ref):
        o = p_ref[0] + p_ref[1]
        m = jnp.max(o, axis=1, keepdims=True)
        ex = jnp.exp(o - m)
        ssum = jnp.sum(ex, axis=1, keepdims=True)
        o_ref[...] = o - m - jnp.log(ssum)

    return pl.pallas_call(
        body,
        out_shape=jax.ShapeDtypeStruct((N, C), jnp.float32),
    )(out2p)


def kernel(x, edge_index, edge_type, edge_attr, W1, W2):
    pad = E1 - E
    src = jnp.concatenate([edge_index[0], jnp.zeros((pad,), jnp.int32)])
    typ = jnp.concatenate([edge_type, jnp.full((pad,), R, jnp.int32)])
    dst = jnp.concatenate([edge_index[1], jnp.zeros((pad,), jnp.int32)])
    attr = jnp.concatenate([edge_attr, jnp.zeros((pad,), jnp.float32)])
    ep = jnp.stack([src, typ, dst, lax.bitcast_convert_type(attr, jnp.int32)])

    yt = _tc_transform1(x, W1)
    out1p, a = _sc_layer1(ep, yt)
    zt = _tc_middle(out1p, W2)
    out2p = _sc_layer2(ep, a, zt)
    return _tc_final(out2p)
